# baseline (speedup 1.0000x reference)
import jax
import jax.numpy as jnp
from jax import lax
from jax.experimental import pallas as pl
from jax.experimental.pallas import tpu as pltpu
from jax.experimental.pallas import tpu_sc as plsc

_NUM_EMB = 1000000
_D = 64
_B = 4096
_H = 50

_INFO = plsc.get_sparse_core_info()
_NC = _INFO.num_cores
_NS = _INFO.num_subcores
_NW = _NC * _NS
_TOTAL = _B * _H
_CH = 128
_NCH_TOT = _TOTAL // _CH
_NCH = _NCH_TOT // _NW
_NBUF = 2
_BC = _B // _CH


def _transform(gbuf, tbuf, off_v, r1, r2):
  rows = [lax.iota(jnp.int32, 16) + g * 16 for g in range(8)]
  offs = [off_v[r1, r2, pl.ds(g * 16, 16)] for g in range(8)]

  @pl.loop(0, _D)
  def _(d):
    d1 = d // 8
    d2 = d - d1 * 8
    for g in range(8):
      v = plsc.load_gather(gbuf, [rows[g], offs[g] + d])
      tbuf[d1, d2, pl.ds(g * 16, 16)] = v


def _body(idx_hbm, t2_hbm, out_hbm, idxr_v, idx2_v, off_v, *rest):
  gbufs = rest[:_NBUF]
  tbufs = rest[_NBUF:2 * _NBUF]
  gsems = rest[2 * _NBUF:3 * _NBUF]
  wsems = rest[3 * _NBUF:4 * _NBUF]

  wid = lax.axis_index("s") * _NC + lax.axis_index("c")
  k0 = wid * _NCH
  p0 = k0 // 8
  loc = k0 - p0 * 8

  pltpu.sync_copy(idx_hbm.at[pl.ds(p0, 7)], idxr_v)

  @pl.loop(0, 56)
  def _(t):
    r1 = t // 8
    r2 = t - r1 * 8
    for g in range(8):
      v = idxr_v[r1, r2, pl.ds(g * 16, 16)]
      idx2_v[r1, r2, pl.ds(g * 16, 16)] = v >> 1
      off_v[r1, r2, pl.ds(g * 16, 16)] = (v & 1) * 64

  def rowsel(c):
    t = loc + c
    r1 = t // 8
    return r1, t - r1 * 8

  def start_gather(c, b):
    r1, r2 = rowsel(c)
    pltpu.async_copy(t2_hbm.at[idx2_v.at[r1, r2]], gbufs[b], gsems[b])

  def wait_gather(c, b):
    r1, r2 = rowsel(c)
    pltpu.make_async_copy(
        t2_hbm.at[idx2_v.at[r1, r2]], gbufs[b], gsems[b]).wait()

  def out_slices(c, b):
    k = k0 + c
    h = k // _BC
    cc = k - h * _BC
    return [(tbufs[b].at[d1], out_hbm.at[h, d1, cc]) for d1 in range(8)]

  def start_write(c, b):
    for src, dst in out_slices(c, b):
      pltpu.async_copy(src, dst, wsems[b])

  def wait_write(c, b):
    for src, dst in out_slices(c, b):
      pltpu.make_async_copy(src, dst, wsems[b]).wait()

  def slot(i, b, first, last):
    wait_gather(i, b)
    if not first:
      wait_write(i - _NBUF, b)
    r1, r2 = rowsel(i)
    _transform(gbufs[b], tbufs[b], off_v, r1, r2)
    start_write(i, b)
    if not last:
      start_gather(i + _NBUF, b)

  for b in range(_NBUF):
    start_gather(b, b)
  for u in range(_NBUF):
    slot(u, u, first=True, last=False)
  @pl.loop(_NBUF, _NCH - _NBUF, step=_NBUF)
  def _(i0):
    for u in range(_NBUF):
      slot(i0 + u, u, first=False, last=False)
  for u in range(_NBUF):
    slot(_NCH - _NBUF + u, u, first=False, last=True)
  for u in range(_NBUF):
    wait_write(_NCH - _NBUF + u, u)


@jax.jit
def _run(idx, t2):
  mesh = plsc.VectorSubcoreMesh(core_axis_name="c", subcore_axis_name="s")
  scratch = (
      [pltpu.VMEM((7, 8, _CH), jnp.int32) for _ in range(3)]
      + [pltpu.VMEM((_CH, _CH), jnp.float32) for _ in range(_NBUF)]
      + [pltpu.VMEM((8, 8, _CH), jnp.float32) for _ in range(_NBUF)]
      + [pltpu.SemaphoreType.DMA for _ in range(2 * _NBUF)]
  )
  out = pl.kernel(
      _body,
      out_type=jax.ShapeDtypeStruct((_H, 8, _BC, 8, _CH), jnp.float32),
      mesh=mesh,
      scratch_types=scratch,
      compiler_params=pltpu.CompilerParams(
          use_tc_tiling_on_sc=True, needs_layout_passes=False),
  )(idx, t2)
  return out


def kernel(indices, table):
  idx = indices.T.astype(jnp.int32).reshape(_NCH_TOT // 8, 8, _CH)
  t2 = table.reshape(_NUM_EMB // 2, 2 * _D)
  out5 = _run(idx, t2)
  return out5.transpose(2, 4, 0, 1, 3).reshape(_B, _H, _D)


# speedup vs baseline: 1.4075x; 1.4075x over previous
import jax
import jax.numpy as jnp
from jax import lax
from jax.experimental import pallas as pl
from jax.experimental.pallas import tpu as pltpu
from jax.experimental.pallas import tpu_sc as plsc

_NUM_EMB = 1000000
_D = 64
_B = 4096
_H = 50

_INFO = plsc.get_sparse_core_info()
_NC = _INFO.num_cores
_NS = _INFO.num_subcores
_NW = _NC * _NS
_TOTAL = _B * _H
_CH = 128
_NCH_TOT = _TOTAL // _CH
_NCH = _NCH_TOT // _NW
_NBUF = 2
_BC = _B // _CH
_C = 2048
_KSH = 244 * _C
_N2 = 245 * _C


def _transform(gbuf, tbuf, off_v, r1, r2):
  rows = [lax.iota(jnp.int32, 16) + g * 16 for g in range(8)]
  offs = [off_v[r1, r2, pl.ds(g * 16, 16)] for g in range(8)]

  @pl.loop(0, 8)
  def _(d1):
    for d2 in range(8):
      d = d1 * 8 + d2
      for g in range(8):
        v = plsc.load_gather(gbuf, [rows[g], offs[g] + d])
        tbuf[d1, d2, pl.ds(g * 16, 16)] = v


def _body(idx_hbm, t2_hbm, out_hbm, idxr_v, idx2_v, off_v, *rest):
  gbufs = rest[:_NBUF]
  tbufs = rest[_NBUF:2 * _NBUF]
  gsems = rest[2 * _NBUF:3 * _NBUF]
  wsems = rest[3 * _NBUF:4 * _NBUF]

  wid = lax.axis_index("s") * _NC + lax.axis_index("c")
  k0 = wid * _NCH
  p0 = k0 // 8
  loc = k0 - p0 * 8

  pltpu.sync_copy(idx_hbm.at[pl.ds(p0, 7)], idxr_v)

  @pl.loop(0, 56)
  def _(t):
    r1 = t // 8
    r2 = t - r1 * 8
    for g in range(8):
      v = idxr_v[r1, r2, pl.ds(g * 16, 16)]
      hi = (v >= _KSH).astype(jnp.int32)
      idx2_v[r1, r2, pl.ds(g * 16, 16)] = v - hi * _KSH
      off_v[r1, r2, pl.ds(g * 16, 16)] = hi * 64

  def rowsel(c):
    t = loc + c
    r1 = t // 8
    return r1, t - r1 * 8

  def start_gather(c, b):
    r1, r2 = rowsel(c)
    pltpu.async_copy(t2_hbm.at[idx2_v.at[r1, r2]], gbufs[b], gsems[b])

  def wait_gather(c, b):
    r1, r2 = rowsel(c)
    pltpu.make_async_copy(
        t2_hbm.at[idx2_v.at[r1, r2]], gbufs[b], gsems[b]).wait()

  def out_slices(c, b):
    k = k0 + c
    h = k // _BC
    cc = k - h * _BC
    return [(tbufs[b].at[d1], out_hbm.at[h, d1, cc]) for d1 in range(8)]

  def start_write(c, b):
    for src, dst in out_slices(c, b):
      pltpu.async_copy(src, dst, wsems[b])

  def wait_write(c, b):
    for src, dst in out_slices(c, b):
      pltpu.make_async_copy(src, dst, wsems[b]).wait()

  def slot(i, b, first, last):
    wait_gather(i, b)
    if not first:
      wait_write(i - _NBUF, b)
    r1, r2 = rowsel(i)
    _transform(gbufs[b], tbufs[b], off_v, r1, r2)
    start_write(i, b)
    if not last:
      start_gather(i + _NBUF, b)

  for b in range(_NBUF):
    start_gather(b, b)
  for u in range(_NBUF):
    slot(u, u, first=True, last=False)
  @pl.loop(_NBUF, _NCH - _NBUF, step=_NBUF)
  def _(i0):
    for u in range(_NBUF):
      slot(i0 + u, u, first=False, last=False)
  for u in range(_NBUF):
    slot(_NCH - _NBUF + u, u, first=False, last=True)
  for u in range(_NBUF):
    wait_write(_NCH - _NBUF + u, u)


@jax.jit
def _run(idx, t2):
  mesh = plsc.VectorSubcoreMesh(core_axis_name="c", subcore_axis_name="s")
  scratch = (
      [pltpu.VMEM((7, 8, _CH), jnp.int32) for _ in range(3)]
      + [pltpu.VMEM((_CH, 2 * _D), jnp.float32) for _ in range(_NBUF)]
      + [pltpu.VMEM((8, 8, _CH), jnp.float32) for _ in range(_NBUF)]
      + [pltpu.SemaphoreType.DMA for _ in range(2 * _NBUF)]
  )
  out = pl.kernel(
      _body,
      out_type=jax.ShapeDtypeStruct((_H, 8, _BC, 8, _CH), jnp.float32),
      mesh=mesh,
      scratch_types=scratch,
      compiler_params=pltpu.CompilerParams(
          use_tc_tiling_on_sc=True, needs_layout_passes=False),
  )(idx, t2)
  return out


def _tc_transpose_body(a_ref, b_ref, out_ref):
  out_ref[...] = jnp.concatenate([a_ref[...].T, b_ref[...].T], axis=1)


def _tc_prep(tt):
  return pl.pallas_call(
      _tc_transpose_body,
      grid=(_N2 // _C,),
      in_specs=[
          pl.BlockSpec((_D, _C), lambda i: (0, i)),
          pl.BlockSpec((_D, _C), lambda i: (0, i + _KSH // _C)),
      ],
      out_specs=pl.BlockSpec((_C, 2 * _D), lambda i: (i, 0)),
      out_shape=jax.ShapeDtypeStruct((_N2, 2 * _D), jnp.float32),
  )(tt, tt)


def kernel(indices, table):
  idx = indices.T.astype(jnp.int32).reshape(_NCH_TOT // 8, 8, _CH)
  t2 = _tc_prep(table.T)
  out5 = _run(idx, t2)
  return out5.transpose(2, 4, 0, 1, 3).reshape(_B, _H, _D)
